# baseline (speedup 1.0000x reference)
import functools

import jax
import jax.numpy as jnp
from jax import lax
from jax.experimental import pallas as pl
from jax.experimental.pallas import tpu as pltpu
from jax.experimental.pallas import tpu_sc as plsc

BATCH = 16384
EMBED_DIM = 32

try:
    _info = plsc.get_sparse_core_info()
    _NC = _info.num_cores
    _NS = _info.num_subcores
except Exception:
    _NC, _NS = 2, 16
_NW = _NC * _NS
_BPW = BATCH // _NW
_CHUNK = 128
_NCHUNK = _BPW // _CHUNK


def _sc_body(uid_hbm, mid_hbm, ut_hbm, mt_hbm, w_hbm, b_hbm, out_hbm,
             uidx, midx, urows, mrows, wv, bv, outv, sem):
    wid = lax.axis_index("s") * _NC + lax.axis_index("c")
    base = wid * _BPW

    pltpu.sync_copy(uid_hbm.at[wid], uidx)
    pltpu.sync_copy(mid_hbm.at[wid], midx)
    pltpu.sync_copy(w_hbm, wv)
    pltpu.sync_copy(b_hbm, bv)

    copies = []
    for c in range(_NCHUNK):
        copies.append(pltpu.async_copy(
            ut_hbm.at[uidx.at[c]], urows.at[pl.ds(c * _CHUNK, _CHUNK)], sem))
        copies.append(pltpu.async_copy(
            mt_hbm.at[midx.at[c]], mrows.at[pl.ds(c * _CHUNK, _CHUNK)], sem))
    for cp in copies:
        cp.wait()

    w0 = wv[0]
    w1 = wv[1]
    w2 = wv[2]
    w3 = wv[3]
    bvec = bv[...]
    lane = lax.iota(jnp.int32, 16)

    def group(g, carry):
        r = bvec
        for j in range(16):
            i = g * 16 + j
            u0 = urows[i, pl.ds(0, 16)]
            u1 = urows[i, pl.ds(16, 16)]
            m0 = mrows[i, pl.ds(0, 16)]
            m1 = mrows[i, pl.ds(16, 16)]
            s = u0 * w0 + u1 * w1 + m0 * w2 + m1 * w3
            tvec = jnp.broadcast_to(jnp.sum(s), (16,))
            r = r + jnp.where(lane == j, tvec, 0.0)
        outv[pl.ds(g * 16, 16)] = r
        return carry

    lax.fori_loop(0, _BPW // 16, group, 0)

    pltpu.sync_copy(outv, out_hbm.at[pl.ds(base, _BPW)])


@jax.jit
def _run(uid3d, mid3d, user_table, movie_table, w4x16, bias16):
    k = functools.partial(
        pl.kernel,
        mesh=plsc.VectorSubcoreMesh(core_axis_name="c", subcore_axis_name="s"),
        out_type=jax.ShapeDtypeStruct((BATCH,), jnp.float32),
        compiler_params=pltpu.CompilerParams(
            needs_layout_passes=False, use_tc_tiling_on_sc=False),
        scratch_types=[
            pltpu.VMEM((_NCHUNK, _CHUNK), jnp.int32),
            pltpu.VMEM((_NCHUNK, _CHUNK), jnp.int32),
            pltpu.VMEM((_BPW, EMBED_DIM), jnp.float32),
            pltpu.VMEM((_BPW, EMBED_DIM), jnp.float32),
            pltpu.VMEM((4, 16), jnp.float32),
            pltpu.VMEM((16,), jnp.float32),
            pltpu.VMEM((_BPW,), jnp.float32),
            pltpu.SemaphoreType.DMA,
        ],
    )(_sc_body)
    return k(uid3d, mid3d, user_table, movie_table, w4x16, bias16)


def kernel(user_ids, movie_ids, user_table, movie_table, fc_w, fc_b):
    uid3d = user_ids.astype(jnp.int32).reshape(_NW, _NCHUNK, _CHUNK)
    mid3d = movie_ids.astype(jnp.int32).reshape(_NW, _NCHUNK, _CHUNK)
    w4x16 = fc_w.reshape(4, 16)
    bias16 = jnp.broadcast_to(fc_b.reshape(()), (16,))
    return _run(uid3d, mid3d, user_table, movie_table, w4x16, bias16)


# speedup vs baseline: 4.1138x; 4.1138x over previous
import functools

import jax
import jax.numpy as jnp
from jax import lax
from jax.experimental import pallas as pl
from jax.experimental.pallas import tpu as pltpu
from jax.experimental.pallas import tpu_sc as plsc

BATCH = 16384
EMBED_DIM = 32

try:
    _info = plsc.get_sparse_core_info()
    _NC = _info.num_cores
    _NS = _info.num_subcores
except Exception:
    _NC, _NS = 2, 16
_NW = _NC * _NS
_BPW = BATCH // _NW
_CHUNK = 128
_NCHUNK = _BPW // _CHUNK

_MV_BLK = 8192


def _mv_body(t_ref, w_ref, o_ref):
    o_ref[...] = jnp.sum(t_ref[...] * w_ref[...], axis=0)


def _matvec(t_t, w):
    d, n = t_t.shape
    grid = (n + _MV_BLK - 1) // _MV_BLK
    return pl.pallas_call(
        _mv_body,
        grid=(grid,),
        in_specs=[
            pl.BlockSpec((d, _MV_BLK), lambda i: (0, i)),
            pl.BlockSpec((d, 1), lambda i: (0, 0)),
        ],
        out_specs=pl.BlockSpec((_MV_BLK,), lambda i: (i,)),
        out_shape=jax.ShapeDtypeStruct((n,), jnp.float32),
    )(t_t, w)


def _sc_body(uid_hbm, mid_hbm, udot_hbm, mdot_hbm, b_hbm, out_hbm,
             uidx, midx, uval, mval, bv, outv, sem):
    wid = lax.axis_index("s") * _NC + lax.axis_index("c")
    base = wid * _BPW

    pltpu.sync_copy(uid_hbm.at[wid], uidx)
    pltpu.sync_copy(mid_hbm.at[wid], midx)
    pltpu.sync_copy(b_hbm, bv)

    copies = []
    for c in range(_NCHUNK):
        copies.append(pltpu.async_copy(udot_hbm.at[uidx.at[c]], uval.at[c], sem))
        copies.append(pltpu.async_copy(mdot_hbm.at[midx.at[c]], mval.at[c], sem))
    for cp in copies:
        cp.wait()

    bvec = bv[...]
    for c in range(_NCHUNK):
        for k in range(_CHUNK // 16):
            v = uval[c, pl.ds(k * 16, 16)] + mval[c, pl.ds(k * 16, 16)] + bvec
            outv[pl.ds(c * _CHUNK + k * 16, 16)] = v

    pltpu.sync_copy(outv, out_hbm.at[pl.ds(base, _BPW)])


@jax.jit
def _run(user_ids, movie_ids, user_table, movie_table, fc_w, fc_b):
    udot = _matvec(user_table.T, fc_w[:EMBED_DIM])
    mdot = _matvec(movie_table.T, fc_w[EMBED_DIM:])
    uid3d = user_ids.astype(jnp.int32).reshape(_NW, _NCHUNK, _CHUNK)
    mid3d = movie_ids.astype(jnp.int32).reshape(_NW, _NCHUNK, _CHUNK)
    bias16 = jnp.broadcast_to(fc_b.reshape(()), (16,))

    g = functools.partial(
        pl.kernel,
        mesh=plsc.VectorSubcoreMesh(core_axis_name="c", subcore_axis_name="s"),
        out_type=jax.ShapeDtypeStruct((BATCH,), jnp.float32),
        compiler_params=pltpu.CompilerParams(
            needs_layout_passes=False, use_tc_tiling_on_sc=False),
        scratch_types=[
            pltpu.VMEM((_NCHUNK, _CHUNK), jnp.int32),
            pltpu.VMEM((_NCHUNK, _CHUNK), jnp.int32),
            pltpu.VMEM((_NCHUNK, _CHUNK), jnp.float32),
            pltpu.VMEM((_NCHUNK, _CHUNK), jnp.float32),
            pltpu.VMEM((16,), jnp.float32),
            pltpu.VMEM((_BPW,), jnp.float32),
            pltpu.SemaphoreType.DMA,
        ],
    )(_sc_body)
    return g(uid3d, mid3d, udot, mdot, bias16)


def kernel(user_ids, movie_ids, user_table, movie_table, fc_w, fc_b):
    return _run(user_ids, movie_ids, user_table, movie_table, fc_w, fc_b)


# speedup vs baseline: 5.5510x; 1.3494x over previous
import functools

import jax
import jax.numpy as jnp
from jax import lax
from jax.experimental import pallas as pl
from jax.experimental.pallas import tpu as pltpu
from jax.experimental.pallas import tpu_sc as plsc

BATCH = 16384
EMBED_DIM = 32

try:
    _info = plsc.get_sparse_core_info()
    _NC = _info.num_cores
    _NS = _info.num_subcores
except Exception:
    _NC, _NS = 2, 16
_NW = _NC * _NS
_BPW = BATCH // _NW
_CHUNK = 128
_NCHUNK = _BPW // _CHUNK

_MV_BLK = 16384


def _mv_body(t_ref, w_ref, o_ref):
    o_ref[...] = lax.dot_general(
        w_ref[...], t_ref[...],
        dimension_numbers=(((0,), (0,)), ((), ())),
        preferred_element_type=jnp.float32,
    )[0]


def _matvec(t_t, w):
    d, n = t_t.shape
    grid = (n + _MV_BLK - 1) // _MV_BLK
    return pl.pallas_call(
        _mv_body,
        grid=(grid,),
        in_specs=[
            pl.BlockSpec((d, _MV_BLK), lambda i: (0, i)),
            pl.BlockSpec((d, 1), lambda i: (0, 0)),
        ],
        out_specs=pl.BlockSpec((_MV_BLK,), lambda i: (i,)),
        out_shape=jax.ShapeDtypeStruct((n,), jnp.float32),
    )(t_t, w)


def _sc_body(uid_hbm, mid_hbm, udot_hbm, mdot_hbm, b_hbm, out_hbm,
             uidx, midx, uval, mval, bv, outv, sem):
    wid = lax.axis_index("s") * _NC + lax.axis_index("c")
    base = wid * _BPW

    pltpu.sync_copy(uid_hbm.at[wid], uidx)
    pltpu.sync_copy(mid_hbm.at[wid], midx)
    pltpu.sync_copy(b_hbm, bv)

    copies = []
    for c in range(_NCHUNK):
        copies.append(pltpu.async_copy(udot_hbm.at[uidx.at[c]], uval.at[c], sem))
        copies.append(pltpu.async_copy(mdot_hbm.at[midx.at[c]], mval.at[c], sem))
    for cp in copies:
        cp.wait()

    bvec = bv[...]
    for c in range(_NCHUNK):
        for k in range(_CHUNK // 16):
            v = uval[c, pl.ds(k * 16, 16)] + mval[c, pl.ds(k * 16, 16)] + bvec
            outv[pl.ds(c * _CHUNK + k * 16, 16)] = v

    pltpu.sync_copy(outv, out_hbm.at[pl.ds(base, _BPW)])


@jax.jit
def _run(user_ids, movie_ids, user_table, movie_table, fc_w, fc_b):
    udot = _matvec(user_table.T, fc_w[:EMBED_DIM])
    mdot = _matvec(movie_table.T, fc_w[EMBED_DIM:])
    uid3d = user_ids.astype(jnp.int32).reshape(_NW, _NCHUNK, _CHUNK)
    mid3d = movie_ids.astype(jnp.int32).reshape(_NW, _NCHUNK, _CHUNK)
    bias16 = jnp.broadcast_to(fc_b.reshape(()), (16,))

    g = functools.partial(
        pl.kernel,
        mesh=plsc.VectorSubcoreMesh(core_axis_name="c", subcore_axis_name="s"),
        out_type=jax.ShapeDtypeStruct((BATCH,), jnp.float32),
        compiler_params=pltpu.CompilerParams(
            needs_layout_passes=False, use_tc_tiling_on_sc=False),
        scratch_types=[
            pltpu.VMEM((_NCHUNK, _CHUNK), jnp.int32),
            pltpu.VMEM((_NCHUNK, _CHUNK), jnp.int32),
            pltpu.VMEM((_NCHUNK, _CHUNK), jnp.float32),
            pltpu.VMEM((_NCHUNK, _CHUNK), jnp.float32),
            pltpu.VMEM((16,), jnp.float32),
            pltpu.VMEM((_BPW,), jnp.float32),
            pltpu.SemaphoreType.DMA,
        ],
    )(_sc_body)
    return g(uid3d, mid3d, udot, mdot, bias16)


def kernel(user_ids, movie_ids, user_table, movie_table, fc_w, fc_b):
    return _run(user_ids, movie_ids, user_table, movie_table, fc_w, fc_b)


# speedup vs baseline: 7.0454x; 1.2692x over previous
import functools

import jax
import jax.numpy as jnp
from jax import lax
from jax.experimental import pallas as pl
from jax.experimental.pallas import tpu as pltpu
from jax.experimental.pallas import tpu_sc as plsc

BATCH = 16384
EMBED_DIM = 32

try:
    _info = plsc.get_sparse_core_info()
    _NC = _info.num_cores
    _NS = _info.num_subcores
except Exception:
    _NC, _NS = 2, 16
_NW = _NC * _NS
_BPW = BATCH // _NW
_CHUNK = 128
_NCHUNK = _BPW // _CHUNK

_MV_BLK = 32768


def _mv_body(t_ref, w_ref, o_ref):
    o_ref[...] = lax.dot_general(
        w_ref[...], t_ref[...],
        dimension_numbers=(((0,), (0,)), ((), ())),
        preferred_element_type=jnp.float32,
    )[0]


def _matvec(t_t, w):
    d, n = t_t.shape
    grid = (n + _MV_BLK - 1) // _MV_BLK
    return pl.pallas_call(
        _mv_body,
        grid=(grid,),
        in_specs=[
            pl.BlockSpec((d, _MV_BLK), lambda i: (0, i)),
            pl.BlockSpec((d, 1), lambda i: (0, 0)),
        ],
        out_specs=pl.BlockSpec((_MV_BLK,), lambda i: (i,)),
        out_shape=jax.ShapeDtypeStruct((n,), jnp.float32),
    )(t_t, w)


def _sc_body(uid_hbm, mid_hbm, udot_hbm, mdot_hbm, b_hbm, out_hbm,
             uidx, midx, uval, mval, bv, outv, sem):
    wid = lax.axis_index("s") * _NC + lax.axis_index("c")
    base = wid * _BPW

    pltpu.sync_copy(uid_hbm.at[wid], uidx)
    pltpu.sync_copy(mid_hbm.at[wid], midx)
    pltpu.sync_copy(b_hbm, bv)

    copies = []
    for c in range(_NCHUNK):
        copies.append(pltpu.async_copy(udot_hbm.at[uidx.at[c]], uval.at[c], sem))
        copies.append(pltpu.async_copy(mdot_hbm.at[midx.at[c]], mval.at[c], sem))
    for cp in copies:
        cp.wait()

    bvec = bv[...]
    for c in range(_NCHUNK):
        for k in range(_CHUNK // 16):
            v = uval[c, pl.ds(k * 16, 16)] + mval[c, pl.ds(k * 16, 16)] + bvec
            outv[pl.ds(c * _CHUNK + k * 16, 16)] = v

    pltpu.sync_copy(outv, out_hbm.at[pl.ds(base, _BPW)])


@jax.jit
def _run(user_ids, movie_ids, user_table, movie_table, fc_w, fc_b):
    udot = _matvec(user_table.T, fc_w[:EMBED_DIM])
    mdot = _matvec(movie_table.T, fc_w[EMBED_DIM:])
    uid3d = user_ids.astype(jnp.int32).reshape(_NW, _NCHUNK, _CHUNK)
    mid3d = movie_ids.astype(jnp.int32).reshape(_NW, _NCHUNK, _CHUNK)
    bias16 = jnp.broadcast_to(fc_b.reshape(()), (16,))

    g = functools.partial(
        pl.kernel,
        mesh=plsc.VectorSubcoreMesh(core_axis_name="c", subcore_axis_name="s"),
        out_type=jax.ShapeDtypeStruct((BATCH,), jnp.float32),
        compiler_params=pltpu.CompilerParams(
            needs_layout_passes=False, use_tc_tiling_on_sc=False),
        scratch_types=[
            pltpu.VMEM((_NCHUNK, _CHUNK), jnp.int32),
            pltpu.VMEM((_NCHUNK, _CHUNK), jnp.int32),
            pltpu.VMEM((_NCHUNK, _CHUNK), jnp.float32),
            pltpu.VMEM((_NCHUNK, _CHUNK), jnp.float32),
            pltpu.VMEM((16,), jnp.float32),
            pltpu.VMEM((_BPW,), jnp.float32),
            pltpu.SemaphoreType.DMA,
        ],
    )(_sc_body)
    return g(uid3d, mid3d, udot, mdot, bias16)


def kernel(user_ids, movie_ids, user_table, movie_table, fc_w, fc_b):
    return _run(user_ids, movie_ids, user_table, movie_table, fc_w, fc_b)


# speedup vs baseline: 7.5349x; 1.0695x over previous
import functools

import jax
import jax.numpy as jnp
from jax import lax
from jax.experimental import pallas as pl
from jax.experimental.pallas import tpu as pltpu
from jax.experimental.pallas import tpu_sc as plsc

BATCH = 16384
EMBED_DIM = 32

try:
    _info = plsc.get_sparse_core_info()
    _NC = _info.num_cores
    _NS = _info.num_subcores
except Exception:
    _NC, _NS = 2, 16
_NW = _NC * _NS
_BPW = BATCH // _NW
_CHUNK = 128
_NCHUNK = _BPW // _CHUNK

_MV_BLK = 65536


def _mv_body(t_ref, w_ref, o_ref):
    o_ref[...] = lax.dot_general(
        w_ref[...], t_ref[...],
        dimension_numbers=(((0,), (0,)), ((), ())),
        preferred_element_type=jnp.float32,
    )[0]


def _matvec(t_t, w):
    d, n = t_t.shape
    grid = (n + _MV_BLK - 1) // _MV_BLK
    return pl.pallas_call(
        _mv_body,
        grid=(grid,),
        in_specs=[
            pl.BlockSpec((d, _MV_BLK), lambda i: (0, i)),
            pl.BlockSpec((d, 1), lambda i: (0, 0)),
        ],
        out_specs=pl.BlockSpec((_MV_BLK,), lambda i: (i,)),
        out_shape=jax.ShapeDtypeStruct((n,), jnp.float32),
    )(t_t, w)


def _sc_body(uid_hbm, mid_hbm, udot_hbm, mdot_hbm, b_hbm, out_hbm,
             uidx, midx, uval, mval, bv, outv, sem):
    wid = lax.axis_index("s") * _NC + lax.axis_index("c")
    base = wid * _BPW

    pltpu.sync_copy(uid_hbm.at[wid], uidx)
    pltpu.sync_copy(mid_hbm.at[wid], midx)
    pltpu.sync_copy(b_hbm, bv)

    copies = []
    for c in range(_NCHUNK):
        copies.append(pltpu.async_copy(udot_hbm.at[uidx.at[c]], uval.at[c], sem))
        copies.append(pltpu.async_copy(mdot_hbm.at[midx.at[c]], mval.at[c], sem))
    for cp in copies:
        cp.wait()

    bvec = bv[...]
    for c in range(_NCHUNK):
        for k in range(_CHUNK // 16):
            v = uval[c, pl.ds(k * 16, 16)] + mval[c, pl.ds(k * 16, 16)] + bvec
            outv[pl.ds(c * _CHUNK + k * 16, 16)] = v

    pltpu.sync_copy(outv, out_hbm.at[pl.ds(base, _BPW)])


@jax.jit
def _run(user_ids, movie_ids, user_table, movie_table, fc_w, fc_b):
    udot = _matvec(user_table.T, fc_w[:EMBED_DIM])
    mdot = _matvec(movie_table.T, fc_w[EMBED_DIM:])
    uid3d = user_ids.astype(jnp.int32).reshape(_NW, _NCHUNK, _CHUNK)
    mid3d = movie_ids.astype(jnp.int32).reshape(_NW, _NCHUNK, _CHUNK)
    bias16 = jnp.broadcast_to(fc_b.reshape(()), (16,))

    g = functools.partial(
        pl.kernel,
        mesh=plsc.VectorSubcoreMesh(core_axis_name="c", subcore_axis_name="s"),
        out_type=jax.ShapeDtypeStruct((BATCH,), jnp.float32),
        compiler_params=pltpu.CompilerParams(
            needs_layout_passes=False, use_tc_tiling_on_sc=False),
        scratch_types=[
            pltpu.VMEM((_NCHUNK, _CHUNK), jnp.int32),
            pltpu.VMEM((_NCHUNK, _CHUNK), jnp.int32),
            pltpu.VMEM((_NCHUNK, _CHUNK), jnp.float32),
            pltpu.VMEM((_NCHUNK, _CHUNK), jnp.float32),
            pltpu.VMEM((16,), jnp.float32),
            pltpu.VMEM((_BPW,), jnp.float32),
            pltpu.SemaphoreType.DMA,
        ],
    )(_sc_body)
    return g(uid3d, mid3d, udot, mdot, bias16)


def kernel(user_ids, movie_ids, user_table, movie_table, fc_w, fc_b):
    return _run(user_ids, movie_ids, user_table, movie_table, fc_w, fc_b)
